# baseline (speedup 1.0000x reference)
import jax
import jax.numpy as jnp
from jax import lax
from jax.experimental import pallas as pl
from jax.experimental.pallas import tpu as pltpu
from jax.experimental.pallas import tpu_sc as plsc

_NF = 26
_CARD = 100000
_EMB = 32
_BATCH = 16384
_N = _BATCH * _NF
_NW = 32
_NPW = _N // _NW
_C = 1664
_NCHUNK = _NPW // _C
_G = 128
_NG = _C // _G
_KPF = _C // _NF


def _sc_body(x_hbm, offp_hbm, bias_hbm, table_hbm, out_hbm,
             idx_v, rows_v, offp_v, bias_v, sem):
    wid = lax.axis_index("s") * 2 + lax.axis_index("c")
    base_w = wid * _NPW
    pltpu.sync_copy(offp_hbm, offp_v)
    pltpu.sync_copy(bias_hbm, bias_v)

    def chunk_body(g, carry):
        base = pl.multiple_of(base_w + g * _C, _C)
        pltpu.sync_copy(x_hbm.at[pl.ds(base, _C)], idx_v)

        def add_off(k, c):
            s = pl.multiple_of(k * 16, 16)
            idx_v[pl.ds(s, 16)] = idx_v[pl.ds(s, 16)] + offp_v[pl.ds(s, 16)]
            return c
        lax.fori_loop(0, _C // 16, add_off, 0, unroll=8)

        copies = [
            pltpu.async_copy(
                table_hbm.at[idx_v.at[pl.ds(j * _G, _G)]],
                rows_v.at[pl.ds(j * _G, _G)],
                sem)
            for j in range(_NG)
        ]
        for c in copies:
            c.wait()

        for f in range(_NF):
            b_lo = bias_v[f, pl.ds(0, 16)]
            b_hi = bias_v[f, pl.ds(16, 16)]

            def add_bias(k, c, f=f, b_lo=b_lo, b_hi=b_hi):
                i = k * _NF + f
                rows_v[i, pl.ds(0, 16)] = rows_v[i, pl.ds(0, 16)] + b_lo
                rows_v[i, pl.ds(16, 16)] = rows_v[i, pl.ds(16, 16)] + b_hi
                return c
            lax.fori_loop(0, _KPF, add_bias, 0, unroll=8)

        pltpu.sync_copy(rows_v, out_hbm.at[pl.ds(base, _C)])
        return carry

    lax.fori_loop(0, _NCHUNK, chunk_body, 0)


def kernel(x, table, bias):
    offp = (jnp.arange(_C, dtype=jnp.int32) % _NF) * _CARD
    x_flat = x.reshape(_N)
    mesh = plsc.VectorSubcoreMesh(core_axis_name="c", subcore_axis_name="s")
    f = pl.kernel(
        _sc_body,
        out_type=jax.ShapeDtypeStruct((_N, _EMB), jnp.float32),
        mesh=mesh,
        compiler_params=pltpu.CompilerParams(use_tc_tiling_on_sc=False),
        scratch_types=[
            pltpu.VMEM((_C,), jnp.int32),
            pltpu.VMEM((_C, _EMB), jnp.float32),
            pltpu.VMEM((_C,), jnp.int32),
            pltpu.VMEM((_NF, _EMB), jnp.float32),
            pltpu.SemaphoreType.DMA,
        ],
    )
    out = f(x_flat, offp, bias, table)
    return out.reshape(_BATCH, _NF, _EMB)
